# initial kernel scaffold (unmeasured)
import jax
import jax.numpy as jnp
from jax import lax
from jax.experimental import pallas as pl
from jax.experimental.pallas import tpu as pltpu


def kernel(
    x,
):
    def body(*refs):
        pass

    out_shape = jax.ShapeDtypeStruct(..., jnp.float32)
    return pl.pallas_call(body, out_shape=out_shape)(...)


# baseline (device time: 242636 ns/iter reference)
import jax
import jax.numpy as jnp
from jax import lax
from jax.experimental import pallas as pl
from jax.experimental.pallas import tpu as pltpu

N_DEV = 32


def kernel(x):
    m_per, n = x.shape

    def body(x_ref, out_ref, send_sems, recv_sems):
        my = lax.axis_index("i")

        barrier_sem = pltpu.get_barrier_semaphore()
        for k in range(1, N_DEV):
            pl.semaphore_signal(
                barrier_sem,
                inc=1,
                device_id=((my + k) % N_DEV,),
                device_id_type=pl.DeviceIdType.MESH,
            )
        pl.semaphore_wait(barrier_sem, N_DEV - 1)

        my_slice = pl.ds(my * m_per, m_per)
        out_ref[my_slice, :] = x_ref[:, :].astype(out_ref.dtype)

        sends = []
        for k in range(1, N_DEV):
            rdma = pltpu.make_async_remote_copy(
                src_ref=out_ref.at[my_slice, :],
                dst_ref=out_ref.at[my_slice, :],
                send_sem=send_sems.at[k - 1],
                recv_sem=recv_sems.at[k - 1],
                device_id=((my + k) % N_DEV,),
                device_id_type=pl.DeviceIdType.MESH,
            )
            rdma.start()
            sends.append(rdma)

        for k in range(1, N_DEV):
            src_dev = (my - k) % N_DEV
            sl = pl.ds(src_dev * m_per, m_per)
            recv = pltpu.make_async_remote_copy(
                src_ref=out_ref.at[sl, :],
                dst_ref=out_ref.at[sl, :],
                send_sem=send_sems.at[k - 1],
                recv_sem=recv_sems.at[k - 1],
                device_id=(src_dev,),
                device_id_type=pl.DeviceIdType.MESH,
            )
            recv.wait_recv()
        for rdma in sends:
            rdma.wait_send()

    return pl.pallas_call(
        body,
        out_shape=jax.ShapeDtypeStruct((N_DEV * m_per, n), jnp.bfloat16),
        in_specs=[pl.BlockSpec(memory_space=pltpu.VMEM)],
        out_specs=pl.BlockSpec(memory_space=pltpu.VMEM),
        scratch_shapes=[
            pltpu.SemaphoreType.DMA((N_DEV - 1,)),
            pltpu.SemaphoreType.DMA((N_DEV - 1,)),
        ],
        compiler_params=pltpu.CompilerParams(collective_id=0),
    )(x)


# device time: 130627 ns/iter; 1.8575x vs baseline; 1.8575x over previous
import numpy as np

import jax
import jax.numpy as jnp
from jax import lax
from jax.experimental import pallas as pl
from jax.experimental.pallas import tpu as pltpu

N_DEV = 32

_RING = np.array(
    [1, 2, 5, 6, 14, 13, 10, 9, 17, 18, 21, 22, 30, 29, 26, 25,
     24, 27, 28, 31, 23, 20, 19, 16, 8, 11, 12, 15, 7, 4, 3, 0],
    dtype=np.int32,
)
_POS = np.zeros(N_DEV, dtype=np.int32)
_POS[_RING] = np.arange(N_DEV, dtype=np.int32)

N_R = N_DEV // 2
N_L = N_DEV // 2 - 1


def kernel(x):
    m_per, n = x.shape

    def body(ring_ref, pos_ref, x_ref, out_ref,
             send_r, recv_r, send_l, recv_l):
        my = lax.axis_index("i")
        r = pos_ref[my]
        right = ring_ref[(r + 1) % N_DEV]
        left = ring_ref[(r + N_DEV - 1) % N_DEV]

        def chunk_at(ring_offset_back):
            return ring_ref[(r + N_DEV - ring_offset_back) % N_DEV]

        def make(idx, s_sem, r_sem, dev):
            sl = pl.ds(idx * m_per, m_per)
            return pltpu.make_async_remote_copy(
                src_ref=out_ref.at[sl, :],
                dst_ref=out_ref.at[sl, :],
                send_sem=s_sem,
                recv_sem=r_sem,
                device_id=(dev,),
                device_id_type=pl.DeviceIdType.MESH,
            )

        barrier_sem = pltpu.get_barrier_semaphore()
        for nbr in (left, right):
            pl.semaphore_signal(
                barrier_sem,
                inc=1,
                device_id=(nbr,),
                device_id_type=pl.DeviceIdType.MESH,
            )
        pl.semaphore_wait(barrier_sem, 2)

        out_ref[pl.ds(my * m_per, m_per), :] = x_ref[:, :].astype(
            out_ref.dtype
        )

        sends = []
        s0r = make(my, send_r.at[0], recv_r.at[0], right)
        s0r.start()
        sends.append(s0r)
        s0l = make(my, send_l.at[0], recv_l.at[0], left)
        s0l.start()
        sends.append(s0l)

        for h in range(N_R):
            cr = chunk_at(h + 1)
            make(cr, send_r.at[h], recv_r.at[h], left).wait_recv()
            if h + 1 < N_R:
                s = make(cr, send_r.at[h + 1], recv_r.at[h + 1], right)
                s.start()
                sends.append(s)
            if h < N_L:
                cl = chunk_at(N_DEV - h - 1)
                make(cl, send_l.at[h], recv_l.at[h], right).wait_recv()
                if h + 1 < N_L:
                    s = make(cl, send_l.at[h + 1], recv_l.at[h + 1], left)
                    s.start()
                    sends.append(s)

        for s in sends:
            s.wait_send()

    ring_tab = jnp.asarray(_RING)
    pos_tab = jnp.asarray(_POS)
    return pl.pallas_call(
        body,
        out_shape=jax.ShapeDtypeStruct((N_DEV * m_per, n), jnp.bfloat16),
        in_specs=[
            pl.BlockSpec(memory_space=pltpu.SMEM),
            pl.BlockSpec(memory_space=pltpu.SMEM),
            pl.BlockSpec(memory_space=pltpu.VMEM),
        ],
        out_specs=pl.BlockSpec(memory_space=pltpu.VMEM),
        scratch_shapes=[
            pltpu.SemaphoreType.DMA((N_R,)),
            pltpu.SemaphoreType.DMA((N_R,)),
            pltpu.SemaphoreType.DMA((N_L,)),
            pltpu.SemaphoreType.DMA((N_L,)),
        ],
        compiler_params=pltpu.CompilerParams(collective_id=0),
    )(ring_tab, pos_tab, x)


# device time: 104537 ns/iter; 2.3211x vs baseline; 1.2496x over previous
import numpy as np

import jax
import jax.numpy as jnp
from jax import lax
from jax.experimental import pallas as pl
from jax.experimental.pallas import tpu as pltpu

N_DEV = 32

_RING = np.array(
    [1, 2, 5, 6, 14, 13, 10, 9, 17, 18, 21, 22, 30, 29, 26, 25,
     24, 27, 28, 31, 23, 20, 19, 16, 8, 11, 12, 15, 7, 4, 3, 0],
    dtype=np.int32,
)
_POS = np.zeros(N_DEV, dtype=np.int32)
_POS[_RING] = np.arange(N_DEV, dtype=np.int32)

N_R = N_DEV // 2
N_L = N_DEV // 2 - 1


def kernel(x):
    m_per, n = x.shape

    def body(ring_ref, pos_ref, x_ref, out_ref,
             send_r, recv_r, send_l, recv_l):
        my = lax.axis_index("i")
        r = pos_ref[my]
        right = ring_ref[(r + 1) % N_DEV]
        left = ring_ref[(r + N_DEV - 1) % N_DEV]

        def chunk_at(ring_offset_back):
            return ring_ref[(r + N_DEV - ring_offset_back) % N_DEV]

        half = m_per // 2

        def make(idx, piece, s_sem, r_sem, dev):
            sl = pl.ds(idx * m_per + piece * half, half)
            return pltpu.make_async_remote_copy(
                src_ref=out_ref.at[sl, :],
                dst_ref=out_ref.at[sl, :],
                send_sem=s_sem,
                recv_sem=r_sem,
                device_id=(dev,),
                device_id_type=pl.DeviceIdType.MESH,
            )

        barrier_sem = pltpu.get_barrier_semaphore()
        for nbr in (left, right):
            pl.semaphore_signal(
                barrier_sem,
                inc=1,
                device_id=(nbr,),
                device_id_type=pl.DeviceIdType.MESH,
            )
        pl.semaphore_wait(barrier_sem, 2)

        out_ref[pl.ds(my * m_per, m_per), :] = x_ref[:, :].astype(
            out_ref.dtype
        )

        sends = []
        for p in range(2):
            s = make(my, p, send_r.at[p], recv_r.at[p], right)
            s.start()
            sends.append(s)
            s = make(my, p, send_l.at[p], recv_l.at[p], left)
            s.start()
            sends.append(s)

        for h in range(N_R):
            cr = chunk_at(h + 1)
            cl = chunk_at(N_DEV - h - 1)
            for p in range(2):
                i = 2 * h + p
                make(cr, p, send_r.at[i], recv_r.at[i], left).wait_recv()
                if h + 1 < N_R:
                    s = make(cr, p, send_r.at[i + 2], recv_r.at[i + 2], right)
                    s.start()
                    sends.append(s)
                if h < N_L:
                    make(cl, p, send_l.at[i], recv_l.at[i], right).wait_recv()
                    if h + 1 < N_L:
                        s = make(
                            cl, p, send_l.at[i + 2], recv_l.at[i + 2], left
                        )
                        s.start()
                        sends.append(s)

        for s in sends:
            s.wait_send()

    ring_tab = jnp.asarray(_RING)
    pos_tab = jnp.asarray(_POS)
    return pl.pallas_call(
        body,
        out_shape=jax.ShapeDtypeStruct((N_DEV * m_per, n), jnp.bfloat16),
        in_specs=[
            pl.BlockSpec(memory_space=pltpu.SMEM),
            pl.BlockSpec(memory_space=pltpu.SMEM),
            pl.BlockSpec(memory_space=pltpu.VMEM),
        ],
        out_specs=pl.BlockSpec(memory_space=pltpu.VMEM),
        scratch_shapes=[
            pltpu.SemaphoreType.DMA((2 * N_R,)),
            pltpu.SemaphoreType.DMA((2 * N_R,)),
            pltpu.SemaphoreType.DMA((2 * N_L,)),
            pltpu.SemaphoreType.DMA((2 * N_L,)),
        ],
        compiler_params=pltpu.CompilerParams(collective_id=0),
    )(ring_tab, pos_tab, x)
